# baseline (speedup 1.0000x reference)
import functools

import jax
import jax.numpy as jnp
from jax import lax
from jax.experimental import pallas as pl
from jax.experimental.pallas import tpu as pltpu
from jax.experimental.pallas import tpu_sc as plsc

BATCH = 16384
EMBED = 64
NCOL = 4
NC = 2
NS = 16
NW = NC * NS
BPW = BATCH // NW
CHUNK = 128
NCH = BPW // CHUNK
LANES = 16


def _sc_embed(sib_flat, landmark_table, theta_table):
  mesh = plsc.VectorSubcoreMesh(core_axis_name="c", subcore_axis_name="s")

  @functools.partial(
      pl.kernel,
      mesh=mesh,
      compiler_params=pltpu.CompilerParams(use_tc_tiling_on_sc=False,
                                           needs_layout_passes=False),
      out_type=jax.ShapeDtypeStruct((BATCH, 2, EMBED), jnp.float32),
      scratch_types=[
          pltpu.VMEM((BPW * NCOL,), jnp.int32),
          pltpu.VMEM((NCH, CHUNK), jnp.int32),
          pltpu.VMEM((NCH, CHUNK), jnp.int32),
          pltpu.VMEM((BPW, EMBED), jnp.float32),
          pltpu.VMEM((BPW, EMBED), jnp.float32),
          pltpu.SemaphoreType.DMA,
          pltpu.SemaphoreType.DMA,
      ],
  )
  def body(sib_hbm, lm_hbm, th_hbm, out_hbm,
           cv, i0_v, i1_v, r0_v, r1_v, gsem, wsem):
    wid = lax.axis_index("s") * NC + lax.axis_index("c")
    base = wid * BPW
    pltpu.sync_copy(sib_hbm.at[pl.ds(base * NCOL, BPW * NCOL)], cv)
    col0 = lax.iota(jnp.int32, LANES) * NCOL
    gathers = []
    for c in range(NCH):
      for j in range(CHUNK // LANES):
        off = (c * CHUNK + j * LANES) * NCOL
        v0 = plsc.load_gather(cv, [col0 + off])
        v1 = plsc.load_gather(cv, [col0 + (off + 1)])
        i0_v[c, pl.ds(j * LANES, LANES)] = v0
        i1_v[c, pl.ds(j * LANES, LANES)] = v1
      gathers.append(pltpu.async_copy(
          lm_hbm.at[i0_v.at[c]], r0_v.at[pl.ds(c * CHUNK, CHUNK)], gsem))
      gathers.append(pltpu.async_copy(
          th_hbm.at[i1_v.at[c]], r1_v.at[pl.ds(c * CHUNK, CHUNK)], gsem))
    for g in gathers:
      g.wait()
    w0 = pltpu.async_copy(r0_v, out_hbm.at[pl.ds(base, BPW), 0], wsem)
    w1 = pltpu.async_copy(r1_v, out_hbm.at[pl.ds(base, BPW), 1], wsem)
    w0.wait()
    w1.wait()

  return body(sib_flat, landmark_table, theta_table)


def kernel(symbolic_instructions_batch, landmark_table, theta_table,
           radius_table):
  sib = symbolic_instructions_batch.astype(jnp.int32).reshape(-1)
  out = _sc_embed(sib, landmark_table, theta_table)
  return out.reshape(BATCH, 2 * EMBED)


# speedup vs baseline: 2.4045x; 2.4045x over previous
import functools

import jax
import jax.numpy as jnp
from jax import lax
from jax.experimental import pallas as pl
from jax.experimental.pallas import tpu as pltpu
from jax.experimental.pallas import tpu_sc as plsc

BATCH = 16384
EMBED = 64
NCOL = 4
NC = 2
NS = 16
NW = NC * NS
BPW = BATCH // NW
CHUNK = 128
NCH = BPW // CHUNK
LANES = 16


def _sc_embed(sib_flat, landmark_table, theta_table):
  mesh = plsc.VectorSubcoreMesh(core_axis_name="c", subcore_axis_name="s")

  @functools.partial(
      pl.kernel,
      mesh=mesh,
      compiler_params=pltpu.CompilerParams(use_tc_tiling_on_sc=False,
                                           needs_layout_passes=False),
      out_type=jax.ShapeDtypeStruct((BATCH, 2 * EMBED), jnp.float32),
      scratch_types=[
          pltpu.VMEM((BPW * NCOL,), jnp.int32),
          pltpu.VMEM((NCH, CHUNK), jnp.int32),
          pltpu.VMEM((NCH, CHUNK), jnp.int32),
          pltpu.VMEM((BPW, EMBED), jnp.float32),
          pltpu.VMEM((BPW, EMBED), jnp.float32),
          pltpu.SemaphoreType.DMA,
          pltpu.SemaphoreType.DMA,
      ],
  )
  def body(sib_hbm, lm_hbm, th_hbm, out_hbm,
           cv, i0_v, i1_v, r0_v, r1_v, gsem, wsem):
    wid = lax.axis_index("s") * NC + lax.axis_index("c")
    base = wid * BPW
    pltpu.sync_copy(sib_hbm.at[pl.ds(base * NCOL, BPW * NCOL)], cv)
    col0 = lax.iota(jnp.int32, LANES) * NCOL
    gathers = []
    for c in range(NCH):
      for j in range(CHUNK // LANES):
        off = (c * CHUNK + j * LANES) * NCOL
        v0 = plsc.load_gather(cv, [col0 + off])
        v1 = plsc.load_gather(cv, [col0 + (off + 1)])
        i0_v[c, pl.ds(j * LANES, LANES)] = v0
        i1_v[c, pl.ds(j * LANES, LANES)] = v1
      gathers.append(pltpu.async_copy(
          lm_hbm.at[i0_v.at[c]], r0_v.at[pl.ds(c * CHUNK, CHUNK)], gsem))
      gathers.append(pltpu.async_copy(
          th_hbm.at[i1_v.at[c]], r1_v.at[pl.ds(c * CHUNK, CHUNK)], gsem))
    for g in gathers:
      g.wait()
    w0 = pltpu.async_copy(
        r0_v, out_hbm.at[pl.ds(base, BPW), pl.ds(0, EMBED)], wsem)
    w1 = pltpu.async_copy(
        r1_v, out_hbm.at[pl.ds(base, BPW), pl.ds(EMBED, EMBED)], wsem)
    w0.wait()
    w1.wait()

  return body(sib_flat, landmark_table, theta_table)


def kernel(symbolic_instructions_batch, landmark_table, theta_table,
           radius_table):
  sib = symbolic_instructions_batch.astype(jnp.int32).reshape(-1)
  return _sc_embed(sib, landmark_table, theta_table)


# speedup vs baseline: 2.4178x; 1.0055x over previous
import functools

import jax
import jax.numpy as jnp
from jax import lax
from jax.experimental import pallas as pl
from jax.experimental.pallas import tpu as pltpu
from jax.experimental.pallas import tpu_sc as plsc

BATCH = 16384
EMBED = 64
NCOL = 4
NC = 2
NS = 16
NW = NC * NS
BPW = BATCH // NW
CHUNK = 128
NCH = BPW // CHUNK
LANES = 16


def _sc_embed(sib_flat, landmark_table, theta_table):
  mesh = plsc.VectorSubcoreMesh(core_axis_name="c", subcore_axis_name="s")

  @functools.partial(
      pl.kernel,
      mesh=mesh,
      compiler_params=pltpu.CompilerParams(use_tc_tiling_on_sc=False,
                                           needs_layout_passes=False,
                                           disable_bounds_checks=True,
                                           disable_semaphore_checks=True),
      out_type=jax.ShapeDtypeStruct((BATCH, 2 * EMBED), jnp.float32),
      scratch_types=[
          pltpu.VMEM((BPW * NCOL,), jnp.int32),
          pltpu.VMEM((NCH, CHUNK), jnp.int32),
          pltpu.VMEM((NCH, CHUNK), jnp.int32),
          pltpu.VMEM((BPW, EMBED), jnp.float32),
          pltpu.VMEM((BPW, EMBED), jnp.float32),
          pltpu.SemaphoreType.DMA,
          pltpu.SemaphoreType.DMA,
      ],
  )
  def body(sib_hbm, lm_hbm, th_hbm, out_hbm,
           cv, i0_v, i1_v, r0_v, r1_v, gsem, wsem):
    wid = lax.axis_index("s") * NC + lax.axis_index("c")
    base = wid * BPW
    pltpu.sync_copy(sib_hbm.at[pl.ds(base * NCOL, BPW * NCOL)], cv)
    col0 = lax.iota(jnp.int32, LANES) * NCOL
    gathers = []
    for c in range(NCH):
      for j in range(CHUNK // LANES):
        off = (c * CHUNK + j * LANES) * NCOL
        v0 = plsc.load_gather(cv, [col0 + off])
        v1 = plsc.load_gather(cv, [col0 + (off + 1)])
        i0_v[c, pl.ds(j * LANES, LANES)] = v0
        i1_v[c, pl.ds(j * LANES, LANES)] = v1
      gathers.append(pltpu.async_copy(
          lm_hbm.at[i0_v.at[c]], r0_v.at[pl.ds(c * CHUNK, CHUNK)], gsem))
      gathers.append(pltpu.async_copy(
          th_hbm.at[i1_v.at[c]], r1_v.at[pl.ds(c * CHUNK, CHUNK)], gsem))
    for g in gathers:
      g.wait()
    w0 = pltpu.async_copy(
        r0_v, out_hbm.at[pl.ds(base, BPW), pl.ds(0, EMBED)], wsem)
    w1 = pltpu.async_copy(
        r1_v, out_hbm.at[pl.ds(base, BPW), pl.ds(EMBED, EMBED)], wsem)
    w0.wait()
    w1.wait()

  return body(sib_flat, landmark_table, theta_table)


def kernel(symbolic_instructions_batch, landmark_table, theta_table,
           radius_table):
  sib = symbolic_instructions_batch.astype(jnp.int32).reshape(-1)
  return _sc_embed(sib, landmark_table, theta_table)


# speedup vs baseline: 3.3974x; 1.4052x over previous
import functools

import jax
import jax.numpy as jnp
from jax import lax
from jax.experimental import pallas as pl
from jax.experimental.pallas import tpu as pltpu
from jax.experimental.pallas import tpu_sc as plsc

BATCH = 16384
EMBED = 64
NC = 2
NS = 16
NW = NC * NS
BPW = BATCH // NW
CHUNK = 128
NCH = BPW // CHUNK


def _sc_embed(idx0, idx1, landmark_table, theta_table):
  mesh = plsc.VectorSubcoreMesh(core_axis_name="c", subcore_axis_name="s")

  @functools.partial(
      pl.kernel,
      mesh=mesh,
      compiler_params=pltpu.CompilerParams(use_tc_tiling_on_sc=False,
                                           needs_layout_passes=False,
                                           disable_bounds_checks=True,
                                           disable_semaphore_checks=True),
      out_type=jax.ShapeDtypeStruct((BATCH, 2 * EMBED), jnp.float32),
      scratch_types=[
          pltpu.VMEM((NCH, CHUNK), jnp.int32),
          pltpu.VMEM((NCH, CHUNK), jnp.int32),
          pltpu.VMEM((BPW, EMBED), jnp.float32),
          pltpu.VMEM((BPW, EMBED), jnp.float32),
          pltpu.SemaphoreType.DMA,
          pltpu.SemaphoreType.DMA,
          pltpu.SemaphoreType.DMA,
      ],
  )
  def body(idx0_hbm, idx1_hbm, lm_hbm, th_hbm, out_hbm,
           i0_v, i1_v, r0_v, r1_v, isem, gsem, wsem):
    wid = lax.axis_index("s") * NC + lax.axis_index("c")
    base = wid * BPW
    iload = []
    for c in range(NCH):
      iload.append(pltpu.async_copy(
          idx0_hbm.at[pl.ds(base + c * CHUNK, CHUNK)], i0_v.at[c], isem))
      iload.append(pltpu.async_copy(
          idx1_hbm.at[pl.ds(base + c * CHUNK, CHUNK)], i1_v.at[c], isem))
    gathers = []
    for c in range(NCH):
      iload[2 * c].wait()
      gathers.append(pltpu.async_copy(
          lm_hbm.at[i0_v.at[c]], r0_v.at[pl.ds(c * CHUNK, CHUNK)], gsem))
      iload[2 * c + 1].wait()
      gathers.append(pltpu.async_copy(
          th_hbm.at[i1_v.at[c]], r1_v.at[pl.ds(c * CHUNK, CHUNK)], gsem))
    for g in gathers:
      g.wait()
    w0 = pltpu.async_copy(
        r0_v, out_hbm.at[pl.ds(base, BPW), pl.ds(0, EMBED)], wsem)
    w1 = pltpu.async_copy(
        r1_v, out_hbm.at[pl.ds(base, BPW), pl.ds(EMBED, EMBED)], wsem)
    w0.wait()
    w1.wait()

  return body(idx0, idx1, landmark_table, theta_table)


def kernel(symbolic_instructions_batch, landmark_table, theta_table,
           radius_table):
  sib = symbolic_instructions_batch.astype(jnp.int32)
  return _sc_embed(sib[:, 0], sib[:, 1], landmark_table, theta_table)


# speedup vs baseline: 3.4266x; 1.0086x over previous
import functools

import jax
import jax.numpy as jnp
from jax import lax
from jax.experimental import pallas as pl
from jax.experimental.pallas import tpu as pltpu
from jax.experimental.pallas import tpu_sc as plsc

BATCH = 16384
VOCAB = 1000
EMBED = 64
NC = 2
NS = 16
NW = NC * NS
BPW = BATCH // NW
CHUNK = 128
NCH = BPW // CHUNK


def _sc_embed(idx0, idx1, table):
  mesh = plsc.VectorSubcoreMesh(core_axis_name="c", subcore_axis_name="s")

  @functools.partial(
      pl.kernel,
      mesh=mesh,
      compiler_params=pltpu.CompilerParams(use_tc_tiling_on_sc=False,
                                           needs_layout_passes=False,
                                           disable_bounds_checks=True,
                                           disable_semaphore_checks=True),
      out_type=jax.ShapeDtypeStruct((BATCH, 2 * EMBED), jnp.float32),
      scratch_types=[
          pltpu.VMEM((BPW,), jnp.int32),
          pltpu.VMEM((BPW,), jnp.int32),
          pltpu.VMEM((BPW, EMBED), jnp.float32),
          pltpu.VMEM((BPW, EMBED), jnp.float32),
          pltpu.SemaphoreType.DMA,
          pltpu.SemaphoreType.DMA,
          pltpu.SemaphoreType.DMA,
      ],
  )
  def body(idx0_hbm, idx1_hbm, tbl_hbm, out_hbm,
           i0_v, i1_v, r0_v, r1_v, isem, gsem, wsem):
    wid = lax.axis_index("s") * NC + lax.axis_index("c")
    base = wid * BPW
    l0 = pltpu.async_copy(idx0_hbm.at[pl.ds(base, BPW)], i0_v, isem)
    l1 = pltpu.async_copy(idx1_hbm.at[pl.ds(base, BPW)], i1_v, isem)
    gathers = []
    l0.wait()
    for c in range(NCH):
      gathers.append(pltpu.async_copy(
          tbl_hbm.at[i0_v.at[pl.ds(c * CHUNK, CHUNK)]],
          r0_v.at[pl.ds(c * CHUNK, CHUNK)], gsem))
    l1.wait()
    for c in range(NCH):
      gathers.append(pltpu.async_copy(
          tbl_hbm.at[i1_v.at[pl.ds(c * CHUNK, CHUNK)]],
          r1_v.at[pl.ds(c * CHUNK, CHUNK)], gsem))
    for g in gathers:
      g.wait()
    w0 = pltpu.async_copy(
        r0_v, out_hbm.at[pl.ds(base, BPW), pl.ds(0, EMBED)], wsem)
    w1 = pltpu.async_copy(
        r1_v, out_hbm.at[pl.ds(base, BPW), pl.ds(EMBED, EMBED)], wsem)
    w0.wait()
    w1.wait()

  return body(idx0, idx1, table)


def kernel(symbolic_instructions_batch, landmark_table, theta_table,
           radius_table):
  sib = symbolic_instructions_batch.astype(jnp.int32)
  table = jnp.concatenate([landmark_table, theta_table], axis=0)
  return _sc_embed(sib[:, 0], sib[:, 1] + VOCAB, table)
